# baseline (speedup 1.0000x reference)
import functools

import jax
import jax.numpy as jnp
from jax import lax
from jax.experimental import pallas as pl
from jax.experimental.pallas import tpu as pltpu
from jax.experimental.pallas import tpu_sc as plsc

_M, _D = 8192, 4096
_NC, _NS, _L = 2, 16, 16
_NW = _NC * _NS
_ROWS_W = _M // _NW
_SLAB = 8
_NSLAB = _ROWS_W // _SLAB

_mesh = plsc.VectorSubcoreMesh(core_axis_name="c", subcore_axis_name="s")


def _process_row(row):
    @plsc.parallel_loop(0, _D, 16, unroll=8)
    def _(i):
        v = row[pl.ds(i, 16)]
        m = jnp.abs(v) > 0.0
        row[pl.ds(i, 16)] = jnp.where(m, jnp.float32(2.0), v)


@functools.partial(
    pl.kernel,
    mesh=_mesh,
    out_type=jax.ShapeDtypeStruct((_M, _D), jnp.float32),
    scratch_types=[
        pltpu.VMEM((3, _SLAB, _D), jnp.float32),
        pltpu.SemaphoreType.DMA,
        pltpu.SemaphoreType.DMA,
        pltpu.SemaphoreType.DMA,
        pltpu.SemaphoreType.DMA,
        pltpu.SemaphoreType.DMA,
        pltpu.SemaphoreType.DMA,
    ],
    compiler_params=pltpu.CompilerParams(use_tc_tiling_on_sc=True),
)
def _sc_mask_set(x_hbm, out_hbm, buf, i0, i1, i2, o0, o1, o2):
    wid = lax.axis_index("s") * _NC + lax.axis_index("c")
    base = wid * _ROWS_W
    isems = (i0, i1, i2)
    osems = (o0, o1, o2)

    def in_copy(ci, b):
        return pltpu.make_async_copy(
            x_hbm.at[pl.ds(base + ci * _SLAB, _SLAB), :], buf.at[b], isems[b]
        )

    def out_copy(ci, b):
        return pltpu.make_async_copy(
            buf.at[b], out_hbm.at[pl.ds(base + ci * _SLAB, _SLAB), :], osems[b]
        )

    for b in range(3):
        in_copy(b, b).start()

    def tri_body(q, carry):
        for u in range(3):
            ci = q * 3 + u
            in_copy(ci, u).wait()

            for r in range(_SLAB):
                _process_row(buf.at[u].at[r])

            out_copy(ci, u).start()

            bp = (u + 2) % 3

            @pl.when(ci >= 1)
            def _():
                out_copy(ci - 1, bp).wait()
                in_copy(ci + 2, bp).start()

        return carry

    lax.fori_loop(0, _NSLAB // 3, tri_body, 0)

    for ci in (_NSLAB - 2, _NSLAB - 1):
        b = ci % 3
        in_copy(ci, b).wait()
        for r in range(_SLAB):
            _process_row(buf.at[b].at[r])

        out_copy(ci, b).start()

    for ci in (_NSLAB - 3, _NSLAB - 2, _NSLAB - 1):
        out_copy(ci, ci % 3).wait()


def kernel(x):
    return _sc_mask_set(x)


# speedup vs baseline: 1.0171x; 1.0171x over previous
import functools

import jax
import jax.numpy as jnp
from jax import lax
from jax.experimental import pallas as pl
from jax.experimental.pallas import tpu as pltpu
from jax.experimental.pallas import tpu_sc as plsc

_M, _D = 8192, 4096
_NC, _NS, _L = 2, 16, 16
_NW = _NC * _NS
_ROWS_W = _M // _NW
_SLAB = 8
_CW = 2048
_CPR = _D // _CW
_NCH = (_ROWS_W // _SLAB) * _CPR
_RB = 4
_P = 3

_mesh = plsc.VectorSubcoreMesh(core_axis_name="c", subcore_axis_name="s")


def _process_row(row, width):
    @plsc.parallel_loop(0, width, 16, unroll=8)
    def _(i):
        v = row[pl.ds(i, 16)]
        m = jnp.abs(v) > 0.0
        row[pl.ds(i, 16)] = jnp.where(m, jnp.float32(2.0), v)


@functools.partial(
    pl.kernel,
    mesh=_mesh,
    out_type=jax.ShapeDtypeStruct((_M, _D), jnp.float32),
    scratch_types=[
        pltpu.VMEM((_RB, _SLAB, _CW), jnp.float32),
        [pltpu.SemaphoreType.DMA] * _RB,
        [pltpu.SemaphoreType.DMA] * _RB,
    ],
    compiler_params=pltpu.CompilerParams(use_tc_tiling_on_sc=True),
)
def _sc_mask_set(x_hbm, out_hbm, buf, isems, osems):
    wid = lax.axis_index("s") * _NC + lax.axis_index("c")
    base = wid * _ROWS_W

    def hbm_slice(hbm, ci):
        tr = ci // _CPR
        ch = ci % _CPR
        return hbm.at[pl.ds(base + tr * _SLAB, _SLAB), pl.ds(ch * _CW, _CW)]

    def in_copy(ci, b):
        return pltpu.make_async_copy(hbm_slice(x_hbm, ci), buf.at[b], isems[b])

    def out_copy(ci, b):
        return pltpu.make_async_copy(buf.at[b], hbm_slice(out_hbm, ci), osems[b])

    for b in range(_P):
        in_copy(b, b).start()

    def ring_body(q, carry):
        for u in range(_RB):
            ci = q * _RB + u
            in_copy(ci, u).wait()
            for r in range(_SLAB):
                _process_row(buf.at[u].at[r], _CW)
            out_copy(ci, u).start()

            bp = (u + _P) % _RB

            @pl.when(ci + _P < _NCH)
            def _():
                @pl.when(ci + _P >= _RB)
                def _():
                    out_copy(ci + _P - _RB, bp).wait()

                in_copy(ci + _P, bp).start()

        return carry

    lax.fori_loop(0, _NCH // _RB, ring_body, 0)

    for k in range(_RB):
        ci = _NCH - _RB + k
        out_copy(ci, ci % _RB).wait()


def kernel(x):
    return _sc_mask_set(x)
